# Initial kernel scaffold; baseline (speedup 1.0000x reference)
import jax
import jax.numpy as jnp
from jax.experimental import pallas as pl


def kernel(pos, x_0, x_1, params, cell_ind_0, cell_ind_1, adj_0_0, adj_0_1, adj_1_1):
    raise NotImplementedError("write your pallas kernel here")


# speedup vs baseline: 2.9689x; 2.9689x over previous
import functools

import jax
import jax.numpy as jnp
from jax import lax
from jax.experimental import pallas as pl
from jax.experimental.pallas import tpu as pltpu
from jax.experimental.pallas import tpu_sc as plsc

N0 = 10000
N1 = 40000
E00 = 320000
E01 = 80000
E11 = 80000
H = 128
NC, NS, LANES = 2, 16, 16
REC_W = 128
WINR = 10000
AGG_R = WINR + LANES
F32 = jnp.float32
I32 = jnp.int32


@functools.lru_cache(maxsize=None)
def _make_pair_gather(E, kb):
    nb = E // kb
    assert nb * kb == E and kb % 8 == 0 and kb <= 128
    mesh = plsc.VectorSubcoreMesh(core_axis_name="c", subcore_axis_name="s")

    def body(tabA, idxA, tabB, idxB, outA, outB, ia_v, ib_v, bufA, bufB, semA, semB):
        c = lax.axis_index("c")
        s = lax.axis_index("s")
        wid = s * NC + c

        @pl.loop(wid, nb, step=NC * NS)
        def _blk(b):
            off = b * kb
            pltpu.sync_copy(idxA.at[pl.ds(off, kb)], ia_v)
            pltpu.sync_copy(idxB.at[pl.ds(off, kb)], ib_v)
            cpA = pltpu.async_copy(tabA.at[ia_v], bufA, semA)
            cpB = pltpu.async_copy(tabB.at[ib_v], bufB, semB)
            cpA.wait()
            cpB.wait()
            pltpu.sync_copy(bufA, outA.at[pl.ds(off, kb)])
            pltpu.sync_copy(bufB, outB.at[pl.ds(off, kb)])

    return pl.kernel(
        body,
        out_type=(jax.ShapeDtypeStruct((E, REC_W), F32),
                  jax.ShapeDtypeStruct((E, REC_W), F32)),
        mesh=mesh,
        scratch_types=[
            pltpu.VMEM((kb,), I32),
            pltpu.VMEM((kb,), I32),
            pltpu.VMEM((kb, REC_W), F32),
            pltpu.VMEM((kb, REC_W), F32),
            pltpu.SemaphoreType.DMA,
            pltpu.SemaphoreType.DMA,
        ],
    )


@functools.lru_cache(maxsize=None)
def _make_msg(E, edge_split, kb=128):
    nb = E // kb
    assert nb * kb == E and kb <= 128
    mesh = plsc.VectorSubcoreMesh(core_axis_name="c", subcore_axis_name="s")
    n_out = 2 if edge_split else 4

    def body(atab, btab, cmat, ridx, sidx, zeros_hbm, out,
             ir_v, is_v, il_v, bufA, bufB, bufC, agg, semA, semB, semC):
        c = lax.axis_index("c")
        s = lax.axis_index("s")

        for wi in range(1 if edge_split else 2):
            if edge_split:
                row_base = 0
                out_blk = c
                start = s * NC + c
                stride = NC * NS
            else:
                win = c * 2 + wi
                row_base = win * WINR
                out_blk = win
                start = s
                stride = NS

            @pl.loop(s, AGG_R // 8, step=NS)
            def _z(j):
                pltpu.sync_copy(zeros_hbm, agg.at[pl.ds(j * 8, 8)])

            plsc.subcore_barrier()

            @pl.loop(start, nb, step=stride)
            def _blk(b):
                off = b * kb
                pltpu.sync_copy(ridx.at[pl.ds(off, kb)], ir_v)
                pltpu.sync_copy(sidx.at[pl.ds(off, kb)], is_v)
                cpA = pltpu.async_copy(atab.at[ir_v], bufA, semA)
                cpB = pltpu.async_copy(btab.at[is_v], bufB, semB)
                cpC = pltpu.async_copy(cmat.at[pl.ds(off, kb)], bufC, semC)
                cpA.wait()
                cpB.wait()
                cpC.wait()

                @pl.loop(0, kb)
                def _row(j):
                    for v in range(H // LANES):
                        sl = pl.ds(v * LANES, LANES)
                        x = bufA[j, sl] + bufB[j, sl] + bufC[j, sl]
                        bufA[j, sl] = x / (1.0 + jnp.exp(-x))

                if edge_split:
                    pltpu.sync_copy(bufA, agg.at[ir_v], add=True)
                else:
                    @pl.loop(0, kb // LANES)
                    def _loc(k):
                        sl = pl.ds(k * LANES, LANES)
                        iv = ir_v[sl] - row_base
                        ok = (iv >= 0) & (iv < WINR)
                        il_v[sl] = jnp.where(ok, iv, WINR)

                    pltpu.sync_copy(bufA, agg.at[il_v], add=True)

            plsc.subcore_barrier()

            @pl.loop(s, WINR // 8, step=NS)
            def _w(j):
                pltpu.sync_copy(agg.at[pl.ds(j * 8, 8)],
                                out.at[pl.ds(out_blk * WINR + j * 8, 8)])

            plsc.subcore_barrier()

    return pl.kernel(
        body,
        out_type=jax.ShapeDtypeStruct((n_out * WINR, H), F32),
        mesh=mesh,
        scratch_types=[
            pltpu.VMEM((kb,), I32),
            pltpu.VMEM((kb,), I32),
            pltpu.VMEM((kb,), I32),
            pltpu.VMEM((kb, H), F32),
            pltpu.VMEM((kb, H), F32),
            pltpu.VMEM((kb, H), F32),
            pltpu.VMEM_SHARED((AGG_R, H), F32),
            pltpu.SemaphoreType.DMA,
            pltpu.SemaphoreType.DMA,
            pltpu.SemaphoreType.DMA,
        ],
    )


def _mm_bias(x, w, b, bn):
    N, K = x.shape
    M = w.shape[1]
    assert N % bn == 0

    def body(x_ref, w_ref, b_ref, o_ref):
        o_ref[...] = (jnp.dot(x_ref[...], w_ref[...],
                              preferred_element_type=F32) + b_ref[...])

    return pl.pallas_call(
        body,
        grid=(N // bn,),
        in_specs=[
            pl.BlockSpec((bn, K), lambda i: (i, 0)),
            pl.BlockSpec((K, M), lambda i: (0, 0)),
            pl.BlockSpec((1, M), lambda i: (0, 0)),
        ],
        out_specs=pl.BlockSpec((bn, M), lambda i: (i, 0)),
        out_shape=jax.ShapeDtypeStruct((N, M), F32),
    )(x, w, b)


def _mm3(x, wcat, bn):
    N, K = x.shape
    assert N % bn == 0

    def body(x_ref, w_ref, o1, o2, o3):
        acc = jnp.dot(x_ref[...], w_ref[...], preferred_element_type=F32)
        o1[...] = acc[:, 0:H]
        o2[...] = acc[:, H:2 * H]
        o3[...] = acc[:, 2 * H:3 * H]

    shp = jax.ShapeDtypeStruct((N, H), F32)
    return pl.pallas_call(
        body,
        grid=(N // bn,),
        in_specs=[
            pl.BlockSpec((bn, K), lambda i: (i, 0)),
            pl.BlockSpec((K, 3 * H), lambda i: (0, 0)),
        ],
        out_specs=[pl.BlockSpec((bn, H), lambda i: (i, 0))] * 3,
        out_shape=(shp, shp, shp),
    )(x, wcat)


def _rec1(pa, pb, bn=2000):
    N = pa.shape[0]

    def body(a_ref, b_ref, o_ref):
        a = a_ref[...]
        b = b_ref[...]
        d = a - b
        diam = jnp.sqrt(jnp.sum(d * d, axis=1, keepdims=True))
        ci = lax.broadcasted_iota(I32, a.shape, 1)
        o_ref[...] = jnp.where(ci < 3, 0.5 * (a + b),
                               jnp.where(ci == 3, diam, 0.0))

    return pl.pallas_call(
        body,
        grid=(N // bn,),
        in_specs=[pl.BlockSpec((bn, REC_W), lambda i: (i, 0))] * 2,
        out_specs=pl.BlockSpec((bn, REC_W), lambda i: (i, 0)),
        out_shape=jax.ShapeDtypeStruct((N, REC_W), F32),
    )(pa, pb)


def _edge_feats(ra, rb):
    d = ra - rb
    ci = lax.broadcasted_iota(I32, d.shape, 1)
    d = jnp.where(ci < 3, d, 0.0)
    distsq = jnp.sum(d * d, axis=1, keepdims=True)
    dist = jnp.sqrt(distsq)
    f1 = ra[:, 3:4]
    f2 = rb[:, 3:4]
    return dist, distsq, f1, f2


def _inv_stats(RA, RB, bn):
    E = RA.shape[0]
    assert E % bn == 0

    def body(a_ref, b_ref, o_ref):
        dist, distsq, f1, f2 = _edge_feats(a_ref[...], b_ref[...])
        ci = lax.broadcasted_iota(I32, (dist.shape[0], 8), 1)
        feat = (jnp.where(ci == 0, dist, 0.0) + jnp.where(ci == 1, f1, 0.0)
                + jnp.where(ci == 2, f2, 0.0) + jnp.where(ci == 4, distsq, 0.0)
                + jnp.where(ci == 5, f1 * f1, 0.0)
                + jnp.where(ci == 6, f2 * f2, 0.0))
        s = jnp.sum(feat, axis=0, keepdims=True)

        @pl.when(pl.program_id(0) == 0)
        def _():
            o_ref[...] = jnp.zeros_like(o_ref)

        o_ref[...] += s

    return pl.pallas_call(
        body,
        grid=(E // bn,),
        in_specs=[pl.BlockSpec((bn, REC_W), lambda i: (i, 0))] * 2,
        out_specs=pl.BlockSpec((1, 8), lambda i: (0, 0)),
        out_shape=jax.ShapeDtypeStruct((1, 8), F32),
    )(RA, RB)


def _inv_norm(RA, RB, stats, bn):
    E = RA.shape[0]
    inv_e = 1.0 / E

    def body(a_ref, b_ref, s_ref, o_ref):
        dist, _, f1, f2 = _edge_feats(a_ref[...], b_ref[...])
        ci = lax.broadcasted_iota(I32, (dist.shape[0], 4), 1)
        feat = (jnp.where(ci == 0, dist, 0.0) + jnp.where(ci == 1, f1, 0.0)
                + jnp.where(ci == 2, f2, 0.0))
        mu = s_ref[0:1, 0:4] * inv_e
        var = s_ref[0:1, 4:8] * inv_e - mu * mu
        o_ref[...] = (feat - mu) * lax.rsqrt(var + 1e-5)

    return pl.pallas_call(
        body,
        grid=(E // bn,),
        in_specs=[pl.BlockSpec((bn, REC_W), lambda i: (i, 0))] * 2
        + [pl.BlockSpec((1, 8), lambda i: (0, 0))],
        out_specs=pl.BlockSpec((bn, 4), lambda i: (i, 0)),
        out_shape=jax.ShapeDtypeStruct((E, 4), F32),
    )(RA, RB, stats)


def _update(h, P, w1, w2, b, bn=1000):
    N = h.shape[0]
    k = P.shape[0] // N
    nb = N // bn
    pspecs = [pl.BlockSpec((bn, H), functools.partial(lambda j, i: (j * nb + i, 0), j))
              for j in range(k)]

    def body(x_ref, w1_ref, w2_ref, b_ref, *rest):
        prefs = rest[:-1]
        o_ref = rest[-1]
        agg = prefs[0][...]
        for p in prefs[1:]:
            agg = agg + p[...]
        x = x_ref[...]
        o_ref[...] = (x + jnp.dot(x, w1_ref[...], preferred_element_type=F32)
                      + jnp.dot(agg, w2_ref[...], preferred_element_type=F32)
                      + b_ref[...])

    return pl.pallas_call(
        body,
        grid=(nb,),
        in_specs=[
            pl.BlockSpec((bn, H), lambda i: (i, 0)),
            pl.BlockSpec((H, H), lambda i: (0, 0)),
            pl.BlockSpec((H, H), lambda i: (0, 0)),
            pl.BlockSpec((1, H), lambda i: (0, 0)),
        ] + pspecs,
        out_specs=pl.BlockSpec((bn, H), lambda i: (i, 0)),
        out_shape=jax.ShapeDtypeStruct((N, H), F32),
    )(h, w1, w2, b, *([P] * k))


def _cmat(inv, wi4, b, bn=2000):
    return _mm_bias(inv, wi4, b, bn)


def kernel(pos, x_0, x_1, params, cell_ind_0, cell_ind_1, adj_0_0, adj_0_1, adj_1_1):
    del cell_ind_0
    pos = pos.astype(F32)
    posp = jnp.concatenate([pos, jnp.zeros((N0, REC_W - 3), F32)], axis=1)
    ia = cell_ind_1[:, 0].astype(I32)
    ib = cell_ind_1[:, 1].astype(I32)
    r00, s00 = adj_0_0[0].astype(I32), adj_0_0[1].astype(I32)
    r01, s01 = adj_0_1[0].astype(I32), adj_0_1[1].astype(I32)
    r11, s11 = adj_1_1[0].astype(I32), adj_1_1[1].astype(I32)
    zeros_hbm = jnp.zeros((8, H), F32)

    PA, PB = _make_pair_gather(N1, 64)(posp, ia, posp, ib)
    rec1 = _rec1(PA, PB)

    RA00, RB00 = _make_pair_gather(E00, 128)(posp, r00, posp, s00)
    RA01, RB01 = _make_pair_gather(E01, 128)(posp, r01, rec1, s01)
    RA11, RB11 = _make_pair_gather(E11, 128)(rec1, r11, rec1, s11)
    inv00 = _inv_norm(RA00, RB00, _inv_stats(RA00, RB00, 2000), 2000)
    inv01 = _inv_norm(RA01, RB01, _inv_stats(RA01, RB01, 2000), 2000)
    inv11 = _inv_norm(RA11, RB11, _inv_stats(RA11, RB11, 2000), 2000)

    emb = params["emb"]
    h0 = _mm_bias(x_0.astype(F32), emb["0"]["W"], emb["0"]["b"][None, :], 1000)
    h1 = _mm_bias(x_1.astype(F32), emb["1"]["W"], emb["1"]["b"][None, :], 1000)

    for layer in params["layers"]:
        w00, b00 = layer["msg"]["0_0"]["W"], layer["msg"]["0_0"]["b"]
        w01, b01 = layer["msg"]["0_1"]["W"], layer["msg"]["0_1"]["b"]
        w11, b11 = layer["msg"]["1_1"]["W"], layer["msg"]["1_1"]["b"]

        wcat0 = jnp.concatenate([w00[:H], w00[H:2 * H], w01[:H]], axis=1)
        A00, B00, A01 = _mm3(h0, wcat0, 1000)
        wcat1 = jnp.concatenate([w01[H:2 * H], w11[:H], w11[H:2 * H]], axis=1)
        B01, A11, B11 = _mm3(h1, wcat1, 1000)

        def wi4(w):
            return jnp.concatenate([w[2 * H:], jnp.zeros((1, H), F32)], axis=0)

        C00 = _cmat(inv00, wi4(w00), b00[None, :])
        C01 = _cmat(inv01, wi4(w01), b01[None, :])
        C11 = _cmat(inv11, wi4(w11), b11[None, :])

        P00 = _make_msg(E00, True)(A00, B00, C00, r00, s00, zeros_hbm)
        P01 = _make_msg(E01, True)(A01, B01, C01, r01, s01, zeros_hbm)
        agg1 = _make_msg(E11, False)(A11, B11, C11, r11, s11, zeros_hbm)

        P0 = jnp.concatenate([P00, P01], axis=0)
        u0, u1 = layer["upd"]["0"], layer["upd"]["1"]
        h0 = _update(h0, P0, u0["W"][:H], u0["W"][H:], u0["b"][None, :])
        h1 = _update(h1, agg1, u1["W"][:H], u1["W"][H:], u1["b"][None, :])

    pre = params["pre"]
    out0 = _mm_bias(h0, pre["0"]["W"], pre["0"]["b"][None, :], 1000)
    out1 = _mm_bias(h1, pre["1"]["W"], pre["1"]["b"][None, :], 1000)
    return jnp.concatenate([out0, out1], axis=0)


# speedup vs baseline: 4.5763x; 1.5414x over previous
import functools

import jax
import jax.numpy as jnp
from jax import lax
from jax.experimental import pallas as pl
from jax.experimental.pallas import tpu as pltpu
from jax.experimental.pallas import tpu_sc as plsc

N0 = 10000
N1 = 40000
E00 = 320000
E01 = 80000
E11 = 80000
H = 128
NC, NS, LANES = 2, 16, 16
REC_W = 128
WINR = 10000
AGG_R = WINR + 80
F32 = jnp.float32
I32 = jnp.int32


@functools.lru_cache(maxsize=None)
def _make_pair_gather(E, kb):
    nb = E // kb
    assert nb * kb == E and kb % 8 == 0 and kb <= 128
    mesh = plsc.VectorSubcoreMesh(core_axis_name="c", subcore_axis_name="s")

    def body(tabA, idxA, tabB, idxB, outA, outB,
             ia0, ib0, ia1, ib1, bA0, bB0, bA1, bB1, sA0, sB0, sA1, sB1):
        c = lax.axis_index("c")
        s = lax.axis_index("s")
        start = s * NC + c
        stride = NC * NS
        ias, ibs = (ia0, ia1), (ib0, ib1)
        bAs, bBs = (bA0, bA1), (bB0, bB1)
        sAs, sBs = (sA0, sA1), (sB0, sB1)

        def fire(slot, b):
            off = b * kb
            pltpu.sync_copy(idxA.at[pl.ds(off, kb)], ias[slot])
            pltpu.sync_copy(idxB.at[pl.ds(off, kb)], ibs[slot])
            pltpu.async_copy(tabA.at[ias[slot]], bAs[slot], sAs[slot])
            pltpu.async_copy(tabB.at[ibs[slot]], bBs[slot], sBs[slot])

        def finish(slot, b):
            off = b * kb
            pltpu.make_async_copy(tabA.at[ias[slot]], bAs[slot], sAs[slot]).wait()
            pltpu.make_async_copy(tabB.at[ibs[slot]], bBs[slot], sBs[slot]).wait()
            pltpu.sync_copy(bAs[slot], outA.at[pl.ds(off, kb)])
            pltpu.sync_copy(bBs[slot], outB.at[pl.ds(off, kb)])

        nloc = (nb - start + stride - 1) // stride

        @pl.when(nloc > 0)
        def _():
            fire(0, start)

        @pl.loop(0, (nloc + 1) // 2)
        def _p(p):
            i0 = 2 * p
            i1 = i0 + 1

            @pl.when(i1 < nloc)
            def _():
                fire(1, start + i1 * stride)

            finish(0, start + i0 * stride)

            @pl.when(i1 + 1 < nloc)
            def _():
                fire(0, start + (i1 + 1) * stride)

            @pl.when(i1 < nloc)
            def _():
                finish(1, start + i1 * stride)

    return pl.kernel(
        body,
        out_type=(jax.ShapeDtypeStruct((E, REC_W), F32),
                  jax.ShapeDtypeStruct((E, REC_W), F32)),
        mesh=mesh,
        scratch_types=[
            pltpu.VMEM((kb,), I32),
            pltpu.VMEM((kb,), I32),
            pltpu.VMEM((kb,), I32),
            pltpu.VMEM((kb,), I32),
            pltpu.VMEM((kb, REC_W), F32),
            pltpu.VMEM((kb, REC_W), F32),
            pltpu.VMEM((kb, REC_W), F32),
            pltpu.VMEM((kb, REC_W), F32),
        ] + [pltpu.SemaphoreType.DMA] * 4,
    )


@functools.lru_cache(maxsize=None)
def _make_msg(E, edge_split, kb=64):
    nb = E // kb
    assert nb * kb == E and kb <= 128
    mesh = plsc.VectorSubcoreMesh(core_axis_name="c", subcore_axis_name="s")
    n_out = 2 if edge_split else 4

    def body(atab, btab, cmat, ridx, sidx, zeros_hbm, out,
             ir0, is0, ir1, is1, il_v, bA0, bB0, bC0, bA1, bB1, bC1, agg,
             sA0, sB0, sC0, sA1, sB1, sC1):
        c = lax.axis_index("c")
        s = lax.axis_index("s")
        irs, iss = (ir0, ir1), (is0, is1)
        bAs, bBs, bCs = (bA0, bA1), (bB0, bB1), (bC0, bC1)
        sAs, sBs, sCs = (sA0, sA1), (sB0, sB1), (sC0, sC1)

        def fire(slot, b):
            off = b * kb
            pltpu.sync_copy(ridx.at[pl.ds(off, kb)], irs[slot])
            pltpu.sync_copy(sidx.at[pl.ds(off, kb)], iss[slot])
            pltpu.async_copy(atab.at[irs[slot]], bAs[slot], sAs[slot])
            pltpu.async_copy(btab.at[iss[slot]], bBs[slot], sBs[slot])
            pltpu.async_copy(cmat.at[pl.ds(off, kb)], bCs[slot], sCs[slot])

        def finish(slot, row_base):
            pltpu.make_async_copy(atab.at[irs[slot]], bAs[slot], sAs[slot]).wait()
            pltpu.make_async_copy(btab.at[iss[slot]], bBs[slot], sBs[slot]).wait()
            pltpu.make_async_copy(cmat.at[pl.ds(0, kb)], bCs[slot], sCs[slot]).wait()
            bA, bB, bC = bAs[slot], bBs[slot], bCs[slot]

            @pl.loop(0, kb)
            def _row(j):
                for v in range(H // LANES):
                    sl = pl.ds(v * LANES, LANES)
                    x = bA[j, sl] + bB[j, sl] + bC[j, sl]
                    bA[j, sl] = x / (1.0 + jnp.exp(-x))

            if edge_split:
                pltpu.sync_copy(bA, agg.at[irs[slot]], add=True)
            else:
                @pl.loop(0, kb // LANES)
                def _loc(k):
                    sl = pl.ds(k * LANES, LANES)
                    iv = irs[slot][sl] - row_base
                    ok = (iv >= 0) & (iv < WINR)
                    il_v[sl] = jnp.where(ok, iv, WINR)

                pltpu.sync_copy(bA, agg.at[il_v], add=True)

        for wi in range(1 if edge_split else 2):
            if edge_split:
                row_base = 0
                out_blk = c
                start = s * NC + c
                stride = NC * NS
            else:
                win = c * 2 + wi
                row_base = win * WINR
                out_blk = win
                start = s
                stride = NS

            @pl.loop(s, AGG_R // 80, step=NS)
            def _z(j):
                pltpu.sync_copy(zeros_hbm, agg.at[pl.ds(j * 80, 80)])

            plsc.subcore_barrier()

            nloc = (nb - start + stride - 1) // stride

            @pl.when(nloc > 0)
            def _():
                fire(0, start)

            @pl.loop(0, (nloc + 1) // 2)
            def _p(p):
                i1 = 2 * p + 1

                @pl.when(i1 < nloc)
                def _():
                    fire(1, start + i1 * stride)

                finish(0, row_base)

                @pl.when(i1 + 1 < nloc)
                def _():
                    fire(0, start + (i1 + 1) * stride)

                @pl.when(i1 < nloc)
                def _():
                    finish(1, row_base)

            plsc.subcore_barrier()

            @pl.loop(s, WINR // 80, step=NS)
            def _w(j):
                pltpu.sync_copy(agg.at[pl.ds(j * 80, 80)],
                                out.at[pl.ds(out_blk * WINR + j * 80, 80)])

            plsc.subcore_barrier()

    return pl.kernel(
        body,
        out_type=jax.ShapeDtypeStruct((n_out * WINR, H), F32),
        mesh=mesh,
        scratch_types=[
            pltpu.VMEM((kb,), I32),
            pltpu.VMEM((kb,), I32),
            pltpu.VMEM((kb,), I32),
            pltpu.VMEM((kb,), I32),
            pltpu.VMEM((kb,), I32),
            pltpu.VMEM((kb, H), F32),
            pltpu.VMEM((kb, H), F32),
            pltpu.VMEM((kb, H), F32),
            pltpu.VMEM((kb, H), F32),
            pltpu.VMEM((kb, H), F32),
            pltpu.VMEM((kb, H), F32),
            pltpu.VMEM_SHARED((AGG_R, H), F32),
        ] + [pltpu.SemaphoreType.DMA] * 6,
    )


def _mm_bias(x, w, b, bn):
    N, K = x.shape
    M = w.shape[1]
    assert N % bn == 0

    def body(x_ref, w_ref, b_ref, o_ref):
        o_ref[...] = (jnp.dot(x_ref[...], w_ref[...],
                              preferred_element_type=F32) + b_ref[...])

    return pl.pallas_call(
        body,
        grid=(N // bn,),
        in_specs=[
            pl.BlockSpec((bn, K), lambda i: (i, 0)),
            pl.BlockSpec((K, M), lambda i: (0, 0)),
            pl.BlockSpec((1, M), lambda i: (0, 0)),
        ],
        out_specs=pl.BlockSpec((bn, M), lambda i: (i, 0)),
        out_shape=jax.ShapeDtypeStruct((N, M), F32),
    )(x, w, b)


def _mm3(x, wcat, bn):
    N, K = x.shape
    assert N % bn == 0

    def body(x_ref, w_ref, o1, o2, o3):
        acc = jnp.dot(x_ref[...], w_ref[...], preferred_element_type=F32)
        o1[...] = acc[:, 0:H]
        o2[...] = acc[:, H:2 * H]
        o3[...] = acc[:, 2 * H:3 * H]

    shp = jax.ShapeDtypeStruct((N, H), F32)
    return pl.pallas_call(
        body,
        grid=(N // bn,),
        in_specs=[
            pl.BlockSpec((bn, K), lambda i: (i, 0)),
            pl.BlockSpec((K, 3 * H), lambda i: (0, 0)),
        ],
        out_specs=[pl.BlockSpec((bn, H), lambda i: (i, 0))] * 3,
        out_shape=(shp, shp, shp),
    )(x, wcat)


def _rec1(pa, pb, bn=2000):
    N = pa.shape[0]

    def body(a_ref, b_ref, o_ref):
        a = a_ref[...]
        b = b_ref[...]
        d = a - b
        diam = jnp.sqrt(jnp.sum(d * d, axis=1, keepdims=True))
        ci = lax.broadcasted_iota(I32, a.shape, 1)
        o_ref[...] = jnp.where(ci < 3, 0.5 * (a + b),
                               jnp.where(ci == 3, diam, 0.0))

    return pl.pallas_call(
        body,
        grid=(N // bn,),
        in_specs=[pl.BlockSpec((bn, REC_W), lambda i: (i, 0))] * 2,
        out_specs=pl.BlockSpec((bn, REC_W), lambda i: (i, 0)),
        out_shape=jax.ShapeDtypeStruct((N, REC_W), F32),
    )(pa, pb)


def _edge_feats(ra, rb):
    d = ra - rb
    ci = lax.broadcasted_iota(I32, d.shape, 1)
    d = jnp.where(ci < 3, d, 0.0)
    distsq = jnp.sum(d * d, axis=1, keepdims=True)
    dist = jnp.sqrt(distsq)
    f1 = ra[:, 3:4]
    f2 = rb[:, 3:4]
    return dist, distsq, f1, f2


def _inv_stats(RA, RB, bn):
    E = RA.shape[0]
    assert E % bn == 0

    def body(a_ref, b_ref, o_ref):
        dist, distsq, f1, f2 = _edge_feats(a_ref[...], b_ref[...])
        ci = lax.broadcasted_iota(I32, (dist.shape[0], 8), 1)
        feat = (jnp.where(ci == 0, dist, 0.0) + jnp.where(ci == 1, f1, 0.0)
                + jnp.where(ci == 2, f2, 0.0) + jnp.where(ci == 4, distsq, 0.0)
                + jnp.where(ci == 5, f1 * f1, 0.0)
                + jnp.where(ci == 6, f2 * f2, 0.0))
        s = jnp.sum(feat, axis=0, keepdims=True)

        @pl.when(pl.program_id(0) == 0)
        def _():
            o_ref[...] = jnp.zeros_like(o_ref)

        o_ref[...] += s

    return pl.pallas_call(
        body,
        grid=(E // bn,),
        in_specs=[pl.BlockSpec((bn, REC_W), lambda i: (i, 0))] * 2,
        out_specs=pl.BlockSpec((1, 8), lambda i: (0, 0)),
        out_shape=jax.ShapeDtypeStruct((1, 8), F32),
    )(RA, RB)


def _inv_norm(RA, RB, stats, bn):
    E = RA.shape[0]
    inv_e = 1.0 / E

    def body(a_ref, b_ref, s_ref, o_ref):
        dist, _, f1, f2 = _edge_feats(a_ref[...], b_ref[...])
        ci = lax.broadcasted_iota(I32, (dist.shape[0], 4), 1)
        feat = (jnp.where(ci == 0, dist, 0.0) + jnp.where(ci == 1, f1, 0.0)
                + jnp.where(ci == 2, f2, 0.0))
        mu = s_ref[0:1, 0:4] * inv_e
        var = s_ref[0:1, 4:8] * inv_e - mu * mu
        o_ref[...] = (feat - mu) * lax.rsqrt(var + 1e-5)

    return pl.pallas_call(
        body,
        grid=(E // bn,),
        in_specs=[pl.BlockSpec((bn, REC_W), lambda i: (i, 0))] * 2
        + [pl.BlockSpec((1, 8), lambda i: (0, 0))],
        out_specs=pl.BlockSpec((bn, 4), lambda i: (i, 0)),
        out_shape=jax.ShapeDtypeStruct((E, 4), F32),
    )(RA, RB, stats)


def _update(h, P, w1, w2, b, bn=1000):
    N = h.shape[0]
    k = P.shape[0] // N
    nb = N // bn
    pspecs = [pl.BlockSpec((bn, H), functools.partial(lambda j, i: (j * nb + i, 0), j))
              for j in range(k)]

    def body(x_ref, w1_ref, w2_ref, b_ref, *rest):
        prefs = rest[:-1]
        o_ref = rest[-1]
        agg = prefs[0][...]
        for p in prefs[1:]:
            agg = agg + p[...]
        x = x_ref[...]
        o_ref[...] = (x + jnp.dot(x, w1_ref[...], preferred_element_type=F32)
                      + jnp.dot(agg, w2_ref[...], preferred_element_type=F32)
                      + b_ref[...])

    return pl.pallas_call(
        body,
        grid=(nb,),
        in_specs=[
            pl.BlockSpec((bn, H), lambda i: (i, 0)),
            pl.BlockSpec((H, H), lambda i: (0, 0)),
            pl.BlockSpec((H, H), lambda i: (0, 0)),
            pl.BlockSpec((1, H), lambda i: (0, 0)),
        ] + pspecs,
        out_specs=pl.BlockSpec((bn, H), lambda i: (i, 0)),
        out_shape=jax.ShapeDtypeStruct((N, H), F32),
    )(h, w1, w2, b, *([P] * k))


def _cmat(inv, wi4, b, bn=2000):
    return _mm_bias(inv, wi4, b, bn)


def kernel(pos, x_0, x_1, params, cell_ind_0, cell_ind_1, adj_0_0, adj_0_1, adj_1_1):
    del cell_ind_0
    pos = pos.astype(F32)
    posp = jnp.concatenate([pos, jnp.zeros((N0, REC_W - 3), F32)], axis=1)
    ia = cell_ind_1[:, 0].astype(I32)
    ib = cell_ind_1[:, 1].astype(I32)
    r00, s00 = adj_0_0[0].astype(I32), adj_0_0[1].astype(I32)
    r01, s01 = adj_0_1[0].astype(I32), adj_0_1[1].astype(I32)
    r11, s11 = adj_1_1[0].astype(I32), adj_1_1[1].astype(I32)
    zeros_hbm = jnp.zeros((80, H), F32)

    PA, PB = _make_pair_gather(N1, 64)(posp, ia, posp, ib)
    rec1 = _rec1(PA, PB)

    RA00, RB00 = _make_pair_gather(E00, 128)(posp, r00, posp, s00)
    RA01, RB01 = _make_pair_gather(E01, 128)(posp, r01, rec1, s01)
    RA11, RB11 = _make_pair_gather(E11, 128)(rec1, r11, rec1, s11)
    inv00 = _inv_norm(RA00, RB00, _inv_stats(RA00, RB00, 2000), 2000)
    inv01 = _inv_norm(RA01, RB01, _inv_stats(RA01, RB01, 2000), 2000)
    inv11 = _inv_norm(RA11, RB11, _inv_stats(RA11, RB11, 2000), 2000)

    emb = params["emb"]
    h0 = _mm_bias(x_0.astype(F32), emb["0"]["W"], emb["0"]["b"][None, :], 1000)
    h1 = _mm_bias(x_1.astype(F32), emb["1"]["W"], emb["1"]["b"][None, :], 1000)

    for layer in params["layers"]:
        w00, b00 = layer["msg"]["0_0"]["W"], layer["msg"]["0_0"]["b"]
        w01, b01 = layer["msg"]["0_1"]["W"], layer["msg"]["0_1"]["b"]
        w11, b11 = layer["msg"]["1_1"]["W"], layer["msg"]["1_1"]["b"]

        wcat0 = jnp.concatenate([w00[:H], w00[H:2 * H], w01[:H]], axis=1)
        A00, B00, A01 = _mm3(h0, wcat0, 1000)
        wcat1 = jnp.concatenate([w01[H:2 * H], w11[:H], w11[H:2 * H]], axis=1)
        B01, A11, B11 = _mm3(h1, wcat1, 1000)

        def wi4(w):
            return jnp.concatenate([w[2 * H:], jnp.zeros((1, H), F32)], axis=0)

        C00 = _cmat(inv00, wi4(w00), b00[None, :])
        C01 = _cmat(inv01, wi4(w01), b01[None, :])
        C11 = _cmat(inv11, wi4(w11), b11[None, :])

        P00 = _make_msg(E00, True)(A00, B00, C00, r00, s00, zeros_hbm)
        P01 = _make_msg(E01, True)(A01, B01, C01, r01, s01, zeros_hbm)
        agg1 = _make_msg(E11, False)(A11, B11, C11, r11, s11, zeros_hbm)

        P0 = jnp.concatenate([P00, P01], axis=0)
        u0, u1 = layer["upd"]["0"], layer["upd"]["1"]
        h0 = _update(h0, P0, u0["W"][:H], u0["W"][H:], u0["b"][None, :])
        h1 = _update(h1, agg1, u1["W"][:H], u1["W"][H:], u1["b"][None, :])

    pre = params["pre"]
    out0 = _mm_bias(h0, pre["0"]["W"], pre["0"]["b"][None, :], 1000)
    out1 = _mm_bias(h1, pre["1"]["W"], pre["1"]["b"][None, :], 1000)
    return jnp.concatenate([out0, out1], axis=0)
